# baseline (speedup 1.0000x reference)
import functools

import jax
import jax.numpy as jnp
from jax import lax
from jax.experimental import pallas as pl
from jax.experimental.pallas import tpu as pltpu
from jax.experimental.pallas import tpu_sc as plsc

T = 2048
D_MODEL = 1024
D_FF = 2048
E = 8
NUM_ROUTED = 7
FF_CHUNK = 2048
N_FF = D_FF // FF_CHUNK
TBLK = 256
NB_R = 22
P = NB_R * TBLK
BLANES = 64
PW = 128

NC = 2
NS = 16
NW = NC * NS
TOK_W = T // NW

NEG = -1e30
RANK_CHUNK = 512


def _router_body(x_ref, wg_ref, pos_ref, prob_ref, be_ref, ba_ref):
    x = x_ref[...]
    logits = lax.dot_general(x, wg_ref[...], (((1,), (1,)), ((), ())),
                             preferred_element_type=jnp.float32)
    lane = lax.broadcasted_iota(jnp.int32, (T, E), 1)
    logits = jnp.where(lane < NUM_ROUTED, logits, NEG)
    v1 = jnp.max(logits, axis=1, keepdims=True)
    i1 = jnp.min(jnp.where(logits >= v1, lane, E), axis=1, keepdims=True)
    l2 = jnp.where(lane == i1, NEG, logits)
    v2 = jnp.max(l2, axis=1, keepdims=True)
    i2 = jnp.min(jnp.where(l2 >= v2, lane, E), axis=1, keepdims=True)
    ed = jnp.exp(v2 - v1)
    z = 1.0 + ed + 1e-12
    prob_ref[0:T, :] = jnp.broadcast_to(1.0 / z, (T, PW))
    prob_ref[T:2 * T, :] = jnp.broadcast_to(ed / z, (T, PW))

    NCH = 2 * T // RANK_CHUNK
    idx = jnp.concatenate([i1, i2], axis=0)
    lane8 = lax.broadcasted_iota(jnp.int32, (RANK_CHUNK, E), 1)
    oh_p = jnp.concatenate(
        [(lane8 == idx[m * RANK_CHUNK:(m + 1) * RANK_CHUNK, :]).astype(jnp.float32)
         for m in range(NCH)], axis=1)

    ones_col = jnp.ones((RANK_CHUNK, 1), jnp.float32)
    cnt_p = lax.dot_general(oh_p, ones_col, (((0,), (0,)), ((), ())),
                            preferred_element_type=jnp.float32)
    rj = lax.broadcasted_iota(jnp.int32, (NCH * E, NCH * E), 0)
    ck = lax.broadcasted_iota(jnp.int32, (NCH * E, NCH * E), 1)
    same_e = ((rj % E) == (ck % E))
    mt = same_e.astype(jnp.float32)
    mc = jnp.logical_and(same_e, (ck // E) < (rj // E)).astype(jnp.float32)
    cnt_sq = jnp.broadcast_to(cnt_p, (NCH * E, NCH * E))
    tot_col = lax.dot_general(mt, cnt_sq, (((1,), (0,)), ((), ())),
                              preferred_element_type=jnp.float32)[:, 0:1]
    car_col = lax.dot_general(mc, cnt_sq, (((1,), (0,)), ((), ())),
                              preferred_element_type=jnp.float32)[:, 0:1]
    cnt_col = tot_col[0:E, :]

    nb_col = jnp.floor((cnt_col + float(TBLK - 1)) * (1.0 / TBLK))
    r8 = lax.broadcasted_iota(jnp.int32, (E, E), 0)
    c8 = lax.broadcasted_iota(jnp.int32, (E, E), 1)
    l8s = (r8 > c8).astype(jnp.float32)
    nb_sq = jnp.broadcast_to(nb_col, (E, E))
    sblk_sq = lax.dot_general(l8s, nb_sq, (((1,), (0,)), ((), ())),
                              preferred_element_type=jnp.float32)
    sblk_col = sblk_sq[:, 0:1]

    rch = lax.broadcasted_iota(jnp.int32, (RANK_CHUNK, RANK_CHUNK), 0)
    cch = lax.broadcasted_iota(jnp.int32, (RANK_CHUNK, RANK_CHUNK), 1)
    ltri = (rch > cch).astype(jnp.float32)
    ranks_p = lax.dot_general(ltri, oh_p, (((1,), (0,)), ((), ())),
                              preferred_element_type=jnp.float32)
    basestart = jnp.broadcast_to(
        sblk_col.reshape(1, E) * float(TBLK), (NCH, E)).reshape(1, NCH * E)
    ranks_p = (ranks_p + car_col.reshape(1, NCH * E) + basestart) * oh_p
    sel = ((lax.broadcasted_iota(jnp.int32, (NCH * E, NCH), 0) // E)
           == lax.broadcasted_iota(jnp.int32, (NCH * E, NCH), 1)).astype(jnp.float32)
    pos_mat = lax.dot_general(ranks_p, sel, (((1,), (0,)), ((), ())),
                              preferred_element_type=jnp.float32)
    for m in range(NCH):
        sl = slice(m * RANK_CHUNK, (m + 1) * RANK_CHUNK)
        pos_ref[sl, :] = pos_mat[:, m:m + 1].astype(jnp.int32)

    bvec = lax.broadcasted_iota(jnp.int32, (E, BLANES), 1).astype(jnp.float32)
    scol32 = jnp.broadcast_to(sblk_col, (E, BLANES))
    routed_e = jnp.sum(jnp.where(scol32 <= bvec, 1.0, 0.0), axis=0, keepdims=True) - 1.0
    routed_e = jnp.broadcast_to(routed_e, (E, BLANES))
    total_nb = jnp.broadcast_to(jnp.sum(nb_col, axis=0, keepdims=True), (E, BLANES))
    be = jnp.minimum(routed_e, float(NUM_ROUTED - 1))
    active = jnp.where(bvec < total_nb, 1.0, 0.0)
    be_ref[...] = be[0:1, :].astype(jnp.int32)
    ba_ref[...] = active[0:1, :].astype(jnp.int32)


def _router(x, wg8, interpret=False):
    return pl.pallas_call(
        _router_body,
        out_shape=(
            jax.ShapeDtypeStruct((2 * T, 1), jnp.int32),
            jax.ShapeDtypeStruct((2 * T, PW), jnp.float32),
            jax.ShapeDtypeStruct((1, BLANES), jnp.int32),
            jax.ShapeDtypeStruct((1, BLANES), jnp.int32),
        ),
        in_specs=[
            pl.BlockSpec((T, D_MODEL), lambda: (0, 0)),
            pl.BlockSpec((E, D_MODEL), lambda: (0, 0)),
        ],
        out_specs=(
            pl.BlockSpec((2 * T, 1), lambda: (0, 0)),
            pl.BlockSpec((2 * T, PW), lambda: (0, 0)),
            pl.BlockSpec((1, BLANES), lambda: (0, 0)),
            pl.BlockSpec((1, BLANES), lambda: (0, 0)),
        ),
        interpret=interpret,
    )(x, wg8)


def _dispatch_body(x_hbm, pos_hbm, prob_hbm, xd_hbm, ps_hbm,
                   xv, i0, i1, prows, sem):
    wid = lax.axis_index("s") * NC + lax.axis_index("c")
    base = wid * TOK_W
    pltpu.sync_copy(x_hbm.at[pl.ds(base, TOK_W)], xv)
    pltpu.sync_copy(pos_hbm.at[pl.ds(base, TOK_W)], i0)
    pltpu.sync_copy(pos_hbm.at[pl.ds(T + base, TOK_W)], i1)
    c0 = pltpu.async_copy(xv, xd_hbm.at[i0], sem)
    c0.wait()
    c1 = pltpu.async_copy(xv, xd_hbm.at[i1], sem)
    c1.wait()

    pltpu.sync_copy(prob_hbm.at[pl.ds(base, TOK_W)], prows)
    cp = pltpu.async_copy(prows, ps_hbm.at[i0], sem)
    cp.wait()
    pltpu.sync_copy(prob_hbm.at[pl.ds(T + base, TOK_W)], prows)
    cp = pltpu.async_copy(prows, ps_hbm.at[i1], sem)
    cp.wait()


def _dispatch(x, posf, probf):
    mesh = plsc.VectorSubcoreMesh(core_axis_name="c", subcore_axis_name="s",
                                  num_cores=NC, num_subcores=NS)
    fn = pl.kernel(
        _dispatch_body,
        out_type=(
            jax.ShapeDtypeStruct((P, D_MODEL), jnp.float32),
            jax.ShapeDtypeStruct((P, PW), jnp.float32),
        ),
        mesh=mesh,
        scratch_types=[
            pltpu.VMEM((TOK_W, D_MODEL), jnp.float32),
            pltpu.VMEM((TOK_W,), jnp.int32),
            pltpu.VMEM((TOK_W,), jnp.int32),
            pltpu.VMEM((TOK_W, PW), jnp.float32),
            pltpu.SemaphoreType.DMA,
        ],
    )
    return fn(x, posf, probf)


def _grouped_body(be_ref, ba_ref, xd_ref, ps_ref, wg_ref, wu_ref, w3_ref,
                  out_ref, *scratch):
    acc_ref = scratch[0] if scratch else None
    c = pl.program_id(0)
    b = pl.program_id(1)

    @pl.when(ba_ref[b] != 0)
    def _():
        xb = xd_ref[...]
        g = lax.dot_general(xb, wg_ref[0], (((1,), (1,)), ((), ())),
                            preferred_element_type=jnp.float32)
        u = lax.dot_general(xb, wu_ref[0], (((1,), (1,)), ((), ())),
                            preferred_element_type=jnp.float32)
        h = g / (1.0 + jnp.exp(-g)) * u
        part = lax.dot_general(h, w3_ref[0], (((1,), (1,)), ((), ())),
                               preferred_element_type=jnp.float32)
        part = part * ps_ref[:, 0:1]

        if N_FF == 1:
            out_ref[...] = part
        else:
            @pl.when(c == 0)
            def _():
                acc_ref[pl.ds(b * TBLK, TBLK), :] = part

            @pl.when(c == N_FF - 1)
            def _():
                out_ref[...] = acc_ref[pl.ds(b * TBLK, TBLK), :] + part


def _grouped(be, ba, xd, psort, w12, w3):
    grid_spec = pltpu.PrefetchScalarGridSpec(
        num_scalar_prefetch=2,
        grid=(N_FF, NB_R),
        in_specs=[
            pl.BlockSpec((TBLK, D_MODEL), lambda c, b, be, ba: (b, 0)),
            pl.BlockSpec((TBLK, PW), lambda c, b, be, ba: (b, 0)),
            pl.BlockSpec((1, FF_CHUNK, D_MODEL), lambda c, b, be, ba: (be[b], c, 0)),
            pl.BlockSpec((1, FF_CHUNK, D_MODEL),
                         lambda c, b, be, ba: (be[b], N_FF + c, 0)),
            pl.BlockSpec((1, D_MODEL, FF_CHUNK), lambda c, b, be, ba: (be[b], 0, c)),
        ],
        out_specs=pl.BlockSpec(
            (TBLK, D_MODEL),
            lambda c, b, be, ba: (jnp.where(c == N_FF - 1, b, 0), 0)),
        scratch_shapes=(
            [] if N_FF == 1 else [pltpu.VMEM((P, D_MODEL), jnp.float32)]),
    )
    return pl.pallas_call(
        _grouped_body,
        grid_spec=grid_spec,
        out_shape=jax.ShapeDtypeStruct((P, D_MODEL), jnp.float32),
    )(be, ba, xd, psort, w12, w12, w3)


def _shared_body(ss_ref, x_ref, wg_ref, wu_ref, w3_ref, out_ref, *scratch):
    acc_ref = scratch[0] if scratch else None
    c = pl.program_id(0)
    i = pl.program_id(1)
    xb = x_ref[...]
    g = lax.dot_general(xb, wg_ref[...], (((1,), (1,)), ((), ())),
                        preferred_element_type=jnp.float32)
    u = lax.dot_general(xb, wu_ref[...], (((1,), (1,)), ((), ())),
                        preferred_element_type=jnp.float32)
    h = g / (1.0 + jnp.exp(-g)) * u
    part = lax.dot_general(h, w3_ref[...], (((1,), (1,)), ((), ())),
                           preferred_element_type=jnp.float32)

    if N_FF == 1:
        out_ref[...] = part * ss_ref[0]
    else:
        @pl.when(c == 0)
        def _():
            acc_ref[pl.ds(i * TBLK, TBLK), :] = part

        @pl.when(c == N_FF - 1)
        def _():
            out_ref[...] = (acc_ref[pl.ds(i * TBLK, TBLK), :] + part) * ss_ref[0]


def _shared(ss1, x, w12s, w3s):
    return pl.pallas_call(
        _shared_body,
        grid=(N_FF, T // TBLK),
        in_specs=[
            pl.BlockSpec(memory_space=pltpu.SMEM),
            pl.BlockSpec((TBLK, D_MODEL), lambda c, i: (i, 0)),
            pl.BlockSpec((FF_CHUNK, D_MODEL), lambda c, i: (c, 0)),
            pl.BlockSpec((FF_CHUNK, D_MODEL), lambda c, i: (N_FF + c, 0)),
            pl.BlockSpec((D_MODEL, FF_CHUNK), lambda c, i: (0, c)),
        ],
        out_specs=pl.BlockSpec(
            (TBLK, D_MODEL), lambda c, i: (jnp.where(c == N_FF - 1, i, 0), 0)),
        scratch_shapes=(
            [] if N_FF == 1 else [pltpu.VMEM((T, D_MODEL), jnp.float32)]),
        out_shape=jax.ShapeDtypeStruct((T, D_MODEL), jnp.float32),
    )(ss1, x, w12s, w12s, w3s)


def _combine_body(yd_hbm, ysh_hbm, pos_hbm, out_hbm, g0, g1, acc, i0, i1, sem0, sem1):
    c = lax.axis_index("c")
    s = lax.axis_index("s")
    wid = s * NC + c
    base = wid * TOK_W
    half = TOK_W // 2
    for h in range(2):
        bh = base + h * half
        pltpu.sync_copy(pos_hbm.at[pl.ds(bh, half)], i0)
        pltpu.sync_copy(pos_hbm.at[pl.ds(T + bh, half)], i1)
        pltpu.sync_copy(ysh_hbm.at[pl.ds(bh, half)], acc)
        d0 = pltpu.async_copy(yd_hbm.at[i0], g0, sem0)
        d1 = pltpu.async_copy(yd_hbm.at[i1], g1, sem1)
        d0.wait()
        d1.wait()

        def body(i, _):
            for k in range(D_MODEL // 16):
                sl = pl.ds(k * 16, 16)
                acc[i, sl] = acc[i, sl] + g0[i, sl] + g1[i, sl]
            return 0

        lax.fori_loop(0, half, body, 0)
        pltpu.sync_copy(acc, out_hbm.at[pl.ds(bh, half)])


def _combine(yd, ysh, posf):
    mesh = plsc.VectorSubcoreMesh(core_axis_name="c", subcore_axis_name="s",
                                  num_cores=NC, num_subcores=NS)
    half = TOK_W // 2
    fn = pl.kernel(
        _combine_body,
        out_type=jax.ShapeDtypeStruct((T, D_MODEL), jnp.float32),
        mesh=mesh,
        scratch_types=[
            pltpu.VMEM((half, D_MODEL), jnp.float32),
            pltpu.VMEM((half, D_MODEL), jnp.float32),
            pltpu.VMEM((half, D_MODEL), jnp.float32),
            pltpu.VMEM((half,), jnp.int32),
            pltpu.VMEM((half,), jnp.int32),
            pltpu.SemaphoreType.DMA,
            pltpu.SemaphoreType.DMA,
        ],
    )
    return fn(yd, ysh, posf)


@functools.partial(jax.jit, static_argnames=("interpret",))
def _moe(x, Wg, W12, W3, W12s, W3s, shared_scale, interpret=False):
    wg8 = jnp.concatenate([Wg, jnp.zeros((1, D_MODEL), jnp.float32)], axis=0)
    ss1 = shared_scale.reshape(1)

    pos, prob, be, ba = _router(x, wg8, interpret=interpret)
    posf = pos.reshape(2 * T)
    bev = be.reshape(BLANES)
    bav = ba.reshape(BLANES)

    ysh = _shared(ss1, x, W12s, W3s)
    xd, psort = _dispatch(x, posf, prob)
    yd = _grouped(bev, bav, xd, psort, W12, W3)
    out = _combine(yd, ysh, posf)
    return out


def kernel(x, Wg, W12, W3, W12s, W3s, shared_scale):
    return _moe(x, Wg, W12, W3, W12s, W3s, shared_scale)


# speedup vs baseline: 1.0116x; 1.0116x over previous
import functools

import jax
import jax.numpy as jnp
from jax import lax
from jax.experimental import pallas as pl
from jax.experimental.pallas import tpu as pltpu
from jax.experimental.pallas import tpu_sc as plsc

T = 2048
D_MODEL = 1024
D_FF = 2048
E = 8
NUM_ROUTED = 7
FF_CHUNK = 2048
N_FF = D_FF // FF_CHUNK
TBLK = 256
NB_R = 22
P = NB_R * TBLK
BLANES = 64
PW = 128

NC = 2
NS = 16
NW = NC * NS
TOK_W = T // NW

NEG = -1e30
RANK_CHUNK = 512


def _router_body(x_ref, wg_ref, pos_ref, prob_ref, be_ref, ba_ref):
    x = x_ref[...]
    logits = lax.dot_general(x, wg_ref[...], (((1,), (1,)), ((), ())),
                             preferred_element_type=jnp.float32)
    lane = lax.broadcasted_iota(jnp.int32, (T, E), 1)
    logits = jnp.where(lane < NUM_ROUTED, logits, NEG)
    v1 = jnp.max(logits, axis=1, keepdims=True)
    i1 = jnp.min(jnp.where(logits >= v1, lane, E), axis=1, keepdims=True)
    l2 = jnp.where(lane == i1, NEG, logits)
    v2 = jnp.max(l2, axis=1, keepdims=True)
    i2 = jnp.min(jnp.where(l2 >= v2, lane, E), axis=1, keepdims=True)
    ed = jnp.exp(v2 - v1)
    z = 1.0 + ed + 1e-12
    prob_ref[0:T, :] = jnp.broadcast_to(1.0 / z, (T, PW))
    prob_ref[T:2 * T, :] = jnp.broadcast_to(ed / z, (T, PW))

    oh1 = (lane == i1).astype(jnp.float32)
    oh2 = (lane == i2).astype(jnp.float32)
    oh = jnp.concatenate([oh1, oh2], axis=0)

    ones_col = jnp.ones((2 * T, 1), jnp.float32)
    cnt_col = lax.dot_general(oh, ones_col, (((0,), (0,)), ((), ())),
                              preferred_element_type=jnp.float32)
    nb_col = jnp.floor((cnt_col + float(TBLK - 1)) * (1.0 / TBLK))
    r8 = lax.broadcasted_iota(jnp.int32, (E, E), 0)
    c8 = lax.broadcasted_iota(jnp.int32, (E, E), 1)
    l8s = (r8 > c8).astype(jnp.float32)
    nb_sq = jnp.broadcast_to(nb_col, (E, E))
    sblk_sq = lax.dot_general(l8s, nb_sq, (((1,), (0,)), ((), ())),
                              preferred_element_type=jnp.float32)
    sblk_col = sblk_sq[:, 0:1]
    base = lax.dot_general(oh, sblk_col, (((1,), (0,)), ((), ())),
                           preferred_element_type=jnp.float32) * float(TBLK)

    rch = lax.broadcasted_iota(jnp.int32, (RANK_CHUNK, RANK_CHUNK), 0)
    cch = lax.broadcasted_iota(jnp.int32, (RANK_CHUNK, RANK_CHUNK), 1)
    ltri = (rch > cch).astype(jnp.float32)
    carry = jnp.zeros((1, E), jnp.float32)
    for m in range(2 * T // RANK_CHUNK):
        sl = slice(m * RANK_CHUNK, (m + 1) * RANK_CHUNK)
        ohm = oh[sl, :]
        ranks = lax.dot_general(ltri, ohm, (((1,), (0,)), ((), ())),
                                preferred_element_type=jnp.float32) + carry
        r_j = jnp.sum(ranks * ohm, axis=1, keepdims=True)
        pos_ref[sl, :] = (base[sl, :] + r_j).astype(jnp.int32)
        carry = carry + jnp.sum(ohm, axis=0, keepdims=True)

    bvec = lax.broadcasted_iota(jnp.int32, (E, BLANES), 1).astype(jnp.float32)
    scol32 = jnp.broadcast_to(sblk_col, (E, BLANES))
    routed_e = jnp.sum(jnp.where(scol32 <= bvec, 1.0, 0.0), axis=0, keepdims=True) - 1.0
    routed_e = jnp.broadcast_to(routed_e, (E, BLANES))
    total_nb = jnp.broadcast_to(jnp.sum(nb_col, axis=0, keepdims=True), (E, BLANES))
    be = jnp.minimum(routed_e, float(NUM_ROUTED - 1))
    active = jnp.where(bvec < total_nb, 1.0, 0.0)
    be_ref[...] = be[0:1, :].astype(jnp.int32)
    ba_ref[...] = active[0:1, :].astype(jnp.int32)


def _router(x, wg8, interpret=False):
    return pl.pallas_call(
        _router_body,
        out_shape=(
            jax.ShapeDtypeStruct((2 * T, 1), jnp.int32),
            jax.ShapeDtypeStruct((2 * T, PW), jnp.float32),
            jax.ShapeDtypeStruct((1, BLANES), jnp.int32),
            jax.ShapeDtypeStruct((1, BLANES), jnp.int32),
        ),
        in_specs=[
            pl.BlockSpec((T, D_MODEL), lambda: (0, 0)),
            pl.BlockSpec((E, D_MODEL), lambda: (0, 0)),
        ],
        out_specs=(
            pl.BlockSpec((2 * T, 1), lambda: (0, 0)),
            pl.BlockSpec((2 * T, PW), lambda: (0, 0)),
            pl.BlockSpec((1, BLANES), lambda: (0, 0)),
            pl.BlockSpec((1, BLANES), lambda: (0, 0)),
        ),
        interpret=interpret,
    )(x, wg8)


def _dispatch_body(x_hbm, pos_hbm, prob_hbm, xd_hbm, ps_hbm,
                   xv, i0, i1, prows, sem):
    wid = lax.axis_index("s") * NC + lax.axis_index("c")
    base = wid * TOK_W
    pltpu.sync_copy(x_hbm.at[pl.ds(base, TOK_W)], xv)
    pltpu.sync_copy(pos_hbm.at[pl.ds(base, TOK_W)], i0)
    pltpu.sync_copy(pos_hbm.at[pl.ds(T + base, TOK_W)], i1)
    c0 = pltpu.async_copy(xv, xd_hbm.at[i0], sem)
    c0.wait()
    c1 = pltpu.async_copy(xv, xd_hbm.at[i1], sem)
    c1.wait()

    pltpu.sync_copy(prob_hbm.at[pl.ds(base, TOK_W)], prows)
    cp = pltpu.async_copy(prows, ps_hbm.at[i0], sem)
    cp.wait()
    pltpu.sync_copy(prob_hbm.at[pl.ds(T + base, TOK_W)], prows)
    cp = pltpu.async_copy(prows, ps_hbm.at[i1], sem)
    cp.wait()


def _dispatch(x, posf, probf):
    mesh = plsc.VectorSubcoreMesh(core_axis_name="c", subcore_axis_name="s",
                                  num_cores=NC, num_subcores=NS)
    fn = pl.kernel(
        _dispatch_body,
        out_type=(
            jax.ShapeDtypeStruct((P, D_MODEL), jnp.float32),
            jax.ShapeDtypeStruct((P, PW), jnp.float32),
        ),
        mesh=mesh,
        scratch_types=[
            pltpu.VMEM((TOK_W, D_MODEL), jnp.float32),
            pltpu.VMEM((TOK_W,), jnp.int32),
            pltpu.VMEM((TOK_W,), jnp.int32),
            pltpu.VMEM((TOK_W, PW), jnp.float32),
            pltpu.SemaphoreType.DMA,
        ],
    )
    return fn(x, posf, probf)


def _grouped_body(be_ref, ba_ref, xd_ref, ps_ref, wg_ref, wu_ref, w3_ref,
                  out_ref, *scratch):
    acc_ref = scratch[0] if scratch else None
    c = pl.program_id(0)
    b = pl.program_id(1)

    @pl.when(ba_ref[b] != 0)
    def _():
        xb = xd_ref[...]
        g = lax.dot_general(xb, wg_ref[0], (((1,), (1,)), ((), ())),
                            preferred_element_type=jnp.float32)
        u = lax.dot_general(xb, wu_ref[0], (((1,), (1,)), ((), ())),
                            preferred_element_type=jnp.float32)
        h = g / (1.0 + jnp.exp(-g)) * u
        part = lax.dot_general(h, w3_ref[0], (((1,), (1,)), ((), ())),
                               preferred_element_type=jnp.float32)
        part = part * ps_ref[:, 0:1]

        if N_FF == 1:
            out_ref[...] = part
        else:
            @pl.when(c == 0)
            def _():
                acc_ref[pl.ds(b * TBLK, TBLK), :] = part

            @pl.when(c == N_FF - 1)
            def _():
                out_ref[...] = acc_ref[pl.ds(b * TBLK, TBLK), :] + part


def _grouped(be, ba, xd, psort, w12, w3):
    grid_spec = pltpu.PrefetchScalarGridSpec(
        num_scalar_prefetch=2,
        grid=(N_FF, NB_R),
        in_specs=[
            pl.BlockSpec((TBLK, D_MODEL), lambda c, b, be, ba: (b, 0)),
            pl.BlockSpec((TBLK, PW), lambda c, b, be, ba: (b, 0)),
            pl.BlockSpec((1, FF_CHUNK, D_MODEL), lambda c, b, be, ba: (be[b], c, 0)),
            pl.BlockSpec((1, FF_CHUNK, D_MODEL),
                         lambda c, b, be, ba: (be[b], N_FF + c, 0)),
            pl.BlockSpec((1, D_MODEL, FF_CHUNK), lambda c, b, be, ba: (be[b], 0, c)),
        ],
        out_specs=pl.BlockSpec(
            (TBLK, D_MODEL),
            lambda c, b, be, ba: (jnp.where(c == N_FF - 1, b, 0), 0)),
        scratch_shapes=(
            [] if N_FF == 1 else [pltpu.VMEM((P, D_MODEL), jnp.float32)]),
    )
    return pl.pallas_call(
        _grouped_body,
        grid_spec=grid_spec,
        out_shape=jax.ShapeDtypeStruct((P, D_MODEL), jnp.float32),
    )(be, ba, xd, psort, w12, w12, w3)


def _shared_body(ss_ref, x_ref, wg_ref, wu_ref, w3_ref, out_ref, *scratch):
    acc_ref = scratch[0] if scratch else None
    c = pl.program_id(0)
    i = pl.program_id(1)
    xb = x_ref[...]
    g = lax.dot_general(xb, wg_ref[...], (((1,), (1,)), ((), ())),
                        preferred_element_type=jnp.float32)
    u = lax.dot_general(xb, wu_ref[...], (((1,), (1,)), ((), ())),
                        preferred_element_type=jnp.float32)
    h = g / (1.0 + jnp.exp(-g)) * u
    part = lax.dot_general(h, w3_ref[...], (((1,), (1,)), ((), ())),
                           preferred_element_type=jnp.float32)

    if N_FF == 1:
        out_ref[...] = part * ss_ref[0]
    else:
        @pl.when(c == 0)
        def _():
            acc_ref[pl.ds(i * TBLK, TBLK), :] = part

        @pl.when(c == N_FF - 1)
        def _():
            out_ref[...] = (acc_ref[pl.ds(i * TBLK, TBLK), :] + part) * ss_ref[0]


def _shared(ss1, x, w12s, w3s):
    return pl.pallas_call(
        _shared_body,
        grid=(N_FF, T // TBLK),
        in_specs=[
            pl.BlockSpec(memory_space=pltpu.SMEM),
            pl.BlockSpec((TBLK, D_MODEL), lambda c, i: (i, 0)),
            pl.BlockSpec((FF_CHUNK, D_MODEL), lambda c, i: (c, 0)),
            pl.BlockSpec((FF_CHUNK, D_MODEL), lambda c, i: (N_FF + c, 0)),
            pl.BlockSpec((D_MODEL, FF_CHUNK), lambda c, i: (0, c)),
        ],
        out_specs=pl.BlockSpec(
            (TBLK, D_MODEL), lambda c, i: (jnp.where(c == N_FF - 1, i, 0), 0)),
        scratch_shapes=(
            [] if N_FF == 1 else [pltpu.VMEM((T, D_MODEL), jnp.float32)]),
        out_shape=jax.ShapeDtypeStruct((T, D_MODEL), jnp.float32),
    )(ss1, x, w12s, w12s, w3s)


def _combine_body(yd_hbm, ysh_hbm, pos_hbm, out_hbm, g0, g1, acc, i0, i1, sem0, sem1):
    c = lax.axis_index("c")
    s = lax.axis_index("s")
    wid = s * NC + c
    base = wid * TOK_W
    half = TOK_W // 2
    for h in range(2):
        bh = base + h * half
        pltpu.sync_copy(pos_hbm.at[pl.ds(bh, half)], i0)
        pltpu.sync_copy(pos_hbm.at[pl.ds(T + bh, half)], i1)
        pltpu.sync_copy(ysh_hbm.at[pl.ds(bh, half)], acc)
        d0 = pltpu.async_copy(yd_hbm.at[i0], g0, sem0)
        d1 = pltpu.async_copy(yd_hbm.at[i1], g1, sem1)
        d0.wait()
        d1.wait()

        def body(i, _):
            for k in range(D_MODEL // 16):
                sl = pl.ds(k * 16, 16)
                acc[i, sl] = acc[i, sl] + g0[i, sl] + g1[i, sl]
            return 0

        lax.fori_loop(0, half, body, 0)
        pltpu.sync_copy(acc, out_hbm.at[pl.ds(bh, half)])


def _combine(yd, ysh, posf):
    mesh = plsc.VectorSubcoreMesh(core_axis_name="c", subcore_axis_name="s",
                                  num_cores=NC, num_subcores=NS)
    half = TOK_W // 2
    fn = pl.kernel(
        _combine_body,
        out_type=jax.ShapeDtypeStruct((T, D_MODEL), jnp.float32),
        mesh=mesh,
        scratch_types=[
            pltpu.VMEM((half, D_MODEL), jnp.float32),
            pltpu.VMEM((half, D_MODEL), jnp.float32),
            pltpu.VMEM((half, D_MODEL), jnp.float32),
            pltpu.VMEM((half,), jnp.int32),
            pltpu.VMEM((half,), jnp.int32),
            pltpu.SemaphoreType.DMA,
            pltpu.SemaphoreType.DMA,
        ],
    )
    return fn(yd, ysh, posf)


@functools.partial(jax.jit, static_argnames=("interpret",))
def _moe(x, Wg, W12, W3, W12s, W3s, shared_scale, interpret=False):
    wg8 = jnp.concatenate([Wg, jnp.zeros((1, D_MODEL), jnp.float32)], axis=0)
    ss1 = shared_scale.reshape(1)

    pos, prob, be, ba = _router(x, wg8, interpret=interpret)
    posf = pos.reshape(2 * T)
    bev = be.reshape(BLANES)
    bav = ba.reshape(BLANES)

    ysh = _shared(ss1, x, W12s, W3s)
    xd, psort = _dispatch(x, posf, prob)
    yd = _grouped(bev, bav, xd, psort, W12, W3)
    out = _combine(yd, ysh, posf)
    return out


def kernel(x, Wg, W12, W3, W12s, W3s, shared_scale):
    return _moe(x, Wg, W12, W3, W12s, W3s, shared_scale)
